# baseline (speedup 1.0000x reference)
import jax
import jax.numpy as jnp
from jax import lax
from jax.experimental import pallas as pl
from jax.experimental.pallas import tpu as pltpu
from jax.experimental.pallas import tpu_sc as plsc

N = 100000
NNZ = 3200000
L = 5
THRESHOLD = 0.01
SLOPE = 5.0

NC, NS = 2, 16
NW = NC * NS
LANES = 16

NP = 100352
NBLK = 26
BLK = 4096
EPW = NBLK * BLK
NNZ_P = NW * EPW
SUB = NP // NS


def _spmv_body(x_hbm, cols_hbm, rows_hbm, vals_hbm, yp_hbm,
               x_sh, y_sh,
               cols_v0, cols_v1, rows_v0, rows_v1, vals_v0, vals_v1,
               xg_v, w_v, zb_v, sem_in0, sem_in1):
    cid = lax.axis_index("c")
    sid = lax.axis_index("s")
    wid = cid * NS + sid

    cols_v = (cols_v0, cols_v1)
    rows_v = (rows_v0, rows_v1)
    vals_v = (vals_v0, vals_v1)
    sem_in = (sem_in0, sem_in1)

    pltpu.sync_copy(x_hbm.at[pl.ds(sid * SUB, SUB)],
                    x_sh.at[pl.ds(sid * SUB, SUB)])

    @pl.loop(0, SUB, step=LANES)
    def _(i):
        zb_v[pl.ds(i, LANES)] = jnp.zeros((LANES,), jnp.float32)

    pltpu.sync_copy(zb_v, y_sh.at[pl.ds(sid * SUB, SUB)])
    plsc.subcore_barrier()

    e_base = wid * EPW

    def stage(slot, bb):
        e0 = e_base + bb * BLK
        pltpu.async_copy(cols_hbm.at[pl.ds(e0, BLK)], cols_v[slot], sem_in[slot])
        pltpu.async_copy(rows_hbm.at[pl.ds(e0, BLK)], rows_v[slot], sem_in[slot])
        pltpu.async_copy(vals_hbm.at[pl.ds(e0, BLK)], vals_v[slot], sem_in[slot])

    def drain_stage(slot, bb):
        e0 = e_base + bb * BLK
        pltpu.make_async_copy(cols_hbm.at[pl.ds(e0, BLK)], cols_v[slot], sem_in[slot]).wait()
        pltpu.make_async_copy(rows_hbm.at[pl.ds(e0, BLK)], rows_v[slot], sem_in[slot]).wait()
        pltpu.make_async_copy(vals_hbm.at[pl.ds(e0, BLK)], vals_v[slot], sem_in[slot]).wait()

    def phase(cur, bb, do_stage):
        nxt = 1 - cur
        drain_stage(cur, bb)
        if do_stage:
            stage(nxt, bb + 1)
        pltpu.sync_copy(x_sh.at[cols_v[cur]], xg_v)

        @pl.loop(0, BLK, step=LANES)
        def _(i):
            w_v[pl.ds(i, LANES)] = vals_v[cur][pl.ds(i, LANES)] * xg_v[pl.ds(i, LANES)]

        pltpu.sync_copy(w_v, y_sh.at[rows_v[cur]], add=True)

    stage(0, 0)
    phase(0, 0, do_stage=True)

    @pl.loop(1, NBLK - 1, step=2)
    def _(bb):
        phase(1, bb, do_stage=True)
        phase(0, bb + 1, do_stage=True)

    phase(1, NBLK - 1, do_stage=False)

    plsc.subcore_barrier()
    pltpu.sync_copy(y_sh.at[pl.ds(sid * SUB, SUB)],
                    yp_hbm.at[cid, pl.ds(sid * SUB, SUB)])


_spmv = pl.kernel(
    _spmv_body,
    out_type=jax.ShapeDtypeStruct((NC, NP), jnp.float32),
    mesh=plsc.VectorSubcoreMesh(core_axis_name="c", subcore_axis_name="s"),
    scratch_types=[
        pltpu.VMEM_SHARED((NP,), jnp.float32),
        pltpu.VMEM_SHARED((NP,), jnp.float32),
        pltpu.VMEM((BLK,), jnp.int32),
        pltpu.VMEM((BLK,), jnp.int32),
        pltpu.VMEM((BLK,), jnp.int32),
        pltpu.VMEM((BLK,), jnp.int32),
        pltpu.VMEM((BLK,), jnp.float32),
        pltpu.VMEM((BLK,), jnp.float32),
        pltpu.VMEM((BLK,), jnp.float32),
        pltpu.VMEM((BLK,), jnp.float32),
        pltpu.VMEM((SUB,), jnp.float32),
        pltpu.SemaphoreType.DMA,
        pltpu.SemaphoreType.DMA,
    ],
)


def _act_body(yp_ref, m_ref, d_ref, o_ref):
    y = yp_ref[0] + yp_ref[1]
    y = jnp.where(y >= THRESHOLD, y, 0.0)
    a = jnp.tanh(jnp.maximum(SLOPE * y, 0.0))
    o_ref[...] = jnp.where(m_ref[...] > 0.0, d_ref[...], a)


_act = pl.pallas_call(
    _act_body,
    out_shape=jax.ShapeDtypeStruct((NP // 128, 128), jnp.float32),
)


def kernel(inputs, values, rows, cols, sensory_idx):
    zeros = jnp.zeros((NP,), jnp.float32)
    mask = zeros.at[sensory_idx].set(1.0)
    drives = [zeros.at[sensory_idx].set(inputs[:, t]) for t in range(L)]

    pad_e = NNZ_P - NNZ
    cols_p = jnp.pad(cols, (0, pad_e))
    rows_p = jnp.pad(rows, (0, pad_e))
    vals_p = jnp.pad(values, (0, pad_e))
    m2 = mask.reshape(NP // 128, 128)

    x = drives[0]
    acts = [x]
    for t in range(1, L):
        yp = _spmv(x, cols_p, rows_p, vals_p)
        xn = _act(yp.reshape(NC, NP // 128, 128), m2,
                  drives[t].reshape(NP // 128, 128))
        x = xn.reshape(NP)
        acts.append(x)
    return jnp.stack([a[:N] for a in acts], axis=1)


# speedup vs baseline: 1.6151x; 1.6151x over previous
import jax
import jax.numpy as jnp
from jax import lax
from jax.experimental import pallas as pl
from jax.experimental.pallas import tpu as pltpu
from jax.experimental.pallas import tpu_sc as plsc

N = 100000
NNZ = 3200000
L = 5
THRESHOLD = 0.01
SLOPE = 5.0

NC, NS = 2, 16
NW = NC * NS
LANES = 16

NP = 100352
NBLK = 100
BLK = 1024
EPW = NBLK * BLK
NNZ_P = NW * EPW
SUB = NP // NS


def _spmv_body(x_hbm, cols_hbm, rows_hbm, vals_hbm, yp_hbm,
               x_sh, y_sh,
               cols_v0, cols_v1, rows_v0, rows_v1, vals_v0, vals_v1,
               xg_v, w_v, zb_v, sem_in0, sem_in1):
    cid = lax.axis_index("c")
    sid = lax.axis_index("s")
    wid = cid * NS + sid

    cols_v = (cols_v0, cols_v1)
    rows_v = (rows_v0, rows_v1)
    vals_v = (vals_v0, vals_v1)
    sem_in = (sem_in0, sem_in1)

    pltpu.sync_copy(x_hbm.at[pl.ds(sid * SUB, SUB)],
                    x_sh.at[pl.ds(sid * SUB, SUB)])

    @pl.loop(0, SUB, step=LANES)
    def _(i):
        zb_v[pl.ds(i, LANES)] = jnp.zeros((LANES,), jnp.float32)

    pltpu.sync_copy(zb_v, y_sh.at[pl.ds(sid * SUB, SUB)])
    plsc.subcore_barrier()

    e_base = wid * EPW

    def stage(slot, bb):
        e0 = e_base + bb * BLK
        pltpu.async_copy(cols_hbm.at[pl.ds(e0, BLK)], cols_v[slot], sem_in[slot])
        pltpu.async_copy(rows_hbm.at[pl.ds(e0, BLK)], rows_v[slot], sem_in[slot])
        pltpu.async_copy(vals_hbm.at[pl.ds(e0, BLK)], vals_v[slot], sem_in[slot])

    def drain_stage(slot, bb):
        e0 = e_base + bb * BLK
        pltpu.make_async_copy(cols_hbm.at[pl.ds(e0, BLK)], cols_v[slot], sem_in[slot]).wait()
        pltpu.make_async_copy(rows_hbm.at[pl.ds(e0, BLK)], rows_v[slot], sem_in[slot]).wait()
        pltpu.make_async_copy(vals_hbm.at[pl.ds(e0, BLK)], vals_v[slot], sem_in[slot]).wait()

    def phase(cur, bb, do_stage):
        nxt = 1 - cur
        drain_stage(cur, bb)
        if do_stage:
            stage(nxt, bb + 1)
        pltpu.sync_copy(x_sh.at[cols_v[cur]], xg_v)

        @pl.loop(0, BLK, step=LANES)
        def _(i):
            w_v[pl.ds(i, LANES)] = vals_v[cur][pl.ds(i, LANES)] * xg_v[pl.ds(i, LANES)]

        pltpu.sync_copy(w_v, y_sh.at[rows_v[cur]], add=True)

    stage(0, 0)
    phase(0, 0, do_stage=True)

    @pl.loop(1, NBLK - 1, step=2)
    def _(bb):
        phase(1, bb, do_stage=True)
        phase(0, bb + 1, do_stage=True)

    phase(1, NBLK - 1, do_stage=False)

    plsc.subcore_barrier()
    pltpu.sync_copy(y_sh.at[pl.ds(sid * SUB, SUB)],
                    yp_hbm.at[cid, pl.ds(sid * SUB, SUB)])


_spmv = pl.kernel(
    _spmv_body,
    out_type=jax.ShapeDtypeStruct((NC, NP), jnp.float32),
    mesh=plsc.VectorSubcoreMesh(core_axis_name="c", subcore_axis_name="s"),
    scratch_types=[
        pltpu.VMEM_SHARED((NP,), jnp.float32),
        pltpu.VMEM_SHARED((NP,), jnp.float32),
        pltpu.VMEM((BLK,), jnp.int32),
        pltpu.VMEM((BLK,), jnp.int32),
        pltpu.VMEM((BLK,), jnp.int32),
        pltpu.VMEM((BLK,), jnp.int32),
        pltpu.VMEM((BLK,), jnp.float32),
        pltpu.VMEM((BLK,), jnp.float32),
        pltpu.VMEM((BLK,), jnp.float32),
        pltpu.VMEM((BLK,), jnp.float32),
        pltpu.VMEM((SUB,), jnp.float32),
        pltpu.SemaphoreType.DMA,
        pltpu.SemaphoreType.DMA,
    ],
)


def _act_body(yp_ref, m_ref, d_ref, o_ref):
    y = yp_ref[0] + yp_ref[1]
    y = jnp.where(y >= THRESHOLD, y, 0.0)
    a = jnp.tanh(jnp.maximum(SLOPE * y, 0.0))
    o_ref[...] = jnp.where(m_ref[...] > 0.0, d_ref[...], a)


_act = pl.pallas_call(
    _act_body,
    out_shape=jax.ShapeDtypeStruct((NP // 128, 128), jnp.float32),
)


def kernel(inputs, values, rows, cols, sensory_idx):
    zeros = jnp.zeros((NP,), jnp.float32)
    mask = zeros.at[sensory_idx].set(1.0)
    drives = [zeros.at[sensory_idx].set(inputs[:, t]) for t in range(L)]

    pad_e = NNZ_P - NNZ
    cols_p = jnp.pad(cols, (0, pad_e))
    rows_p = jnp.pad(rows, (0, pad_e))
    vals_p = jnp.pad(values, (0, pad_e))
    m2 = mask.reshape(NP // 128, 128)

    x = drives[0]
    acts = [x]
    for t in range(1, L):
        yp = _spmv(x, cols_p, rows_p, vals_p)
        xn = _act(yp.reshape(NC, NP // 128, 128), m2,
                  drives[t].reshape(NP // 128, 128))
        x = xn.reshape(NP)
        acts.append(x)
    return jnp.stack([a[:N] for a in acts], axis=1)


# speedup vs baseline: 1.9319x; 1.1962x over previous
import jax
import jax.numpy as jnp
from jax import lax
from jax.experimental import pallas as pl
from jax.experimental.pallas import tpu as pltpu
from jax.experimental.pallas import tpu_sc as plsc

N = 100000
NNZ = 3200000
L = 5
THRESHOLD = 0.01
SLOPE = 5.0

NC, NS = 2, 16
NW = NC * NS
LANES = 16

NP = 100352
NBLK = 50
BLK = 2048
EPW = NBLK * BLK
NNZ_P = NW * EPW
SUB = NP // NS


def _spmv_body(x_hbm, cols_hbm, rows_hbm, vals_hbm, yp_hbm,
               x_sh, y_sh,
               cols_v0, cols_v1, rows_v0, rows_v1, vals_v0, vals_v1,
               xg_v0, xg_v1, w_v, zb_v, sem_in0, sem_in1, sem_g):
    cid = lax.axis_index("c")
    sid = lax.axis_index("s")
    wid = cid * NS + sid

    cols_v = (cols_v0, cols_v1)
    rows_v = (rows_v0, rows_v1)
    vals_v = (vals_v0, vals_v1)
    sem_in = (sem_in0, sem_in1)

    pltpu.sync_copy(x_hbm.at[pl.ds(sid * SUB, SUB)],
                    x_sh.at[pl.ds(sid * SUB, SUB)])

    @pl.loop(0, SUB, step=LANES)
    def _(i):
        zb_v[pl.ds(i, LANES)] = jnp.zeros((LANES,), jnp.float32)

    pltpu.sync_copy(zb_v, y_sh.at[pl.ds(sid * SUB, SUB)])
    plsc.subcore_barrier()

    e_base = wid * EPW

    def stage(slot, bb):
        e0 = e_base + bb * BLK
        pltpu.async_copy(cols_hbm.at[pl.ds(e0, BLK)], cols_v[slot], sem_in[slot])
        pltpu.async_copy(rows_hbm.at[pl.ds(e0, BLK)], rows_v[slot], sem_in[slot])
        pltpu.async_copy(vals_hbm.at[pl.ds(e0, BLK)], vals_v[slot], sem_in[slot])

    def drain_stage(slot, bb):
        e0 = e_base + bb * BLK
        pltpu.make_async_copy(cols_hbm.at[pl.ds(e0, BLK)], cols_v[slot], sem_in[slot]).wait()
        pltpu.make_async_copy(rows_hbm.at[pl.ds(e0, BLK)], rows_v[slot], sem_in[slot]).wait()
        pltpu.make_async_copy(vals_hbm.at[pl.ds(e0, BLK)], vals_v[slot], sem_in[slot]).wait()

    xg_v = (xg_v0, xg_v1)

    def wait_gather(slot):
        pltpu.make_async_copy(x_sh.at[cols_v[slot]], xg_v[slot], sem_g).wait()

    def fire_gather(slot):
        pltpu.make_async_copy(x_sh.at[cols_v[slot]], xg_v[slot], sem_g).start()

    def phase(cur, bb, do_next, do_stage2):
        nxt = 1 - cur
        wait_gather(cur)
        if do_next:
            drain_stage(nxt, bb + 1)
            fire_gather(nxt)

        @pl.loop(0, BLK, step=LANES)
        def _(i):
            w_v[pl.ds(i, LANES)] = vals_v[cur][pl.ds(i, LANES)] * xg_v[cur][pl.ds(i, LANES)]

        pltpu.sync_copy(w_v, y_sh.at[rows_v[cur]], add=True)
        if do_stage2:
            stage(cur, bb + 2)

    stage(0, 0)
    stage(1, 1)
    drain_stage(0, 0)
    fire_gather(0)
    phase(0, 0, do_next=True, do_stage2=True)

    @pl.loop(1, NBLK - 3, step=2)
    def _(bb):
        phase(1, bb, do_next=True, do_stage2=True)
        phase(0, bb + 1, do_next=True, do_stage2=True)

    phase(1, NBLK - 3, do_next=True, do_stage2=True)
    phase(0, NBLK - 2, do_next=True, do_stage2=False)
    phase(1, NBLK - 1, do_next=False, do_stage2=False)

    plsc.subcore_barrier()
    pltpu.sync_copy(y_sh.at[pl.ds(sid * SUB, SUB)],
                    yp_hbm.at[cid, pl.ds(sid * SUB, SUB)])


_spmv = pl.kernel(
    _spmv_body,
    out_type=jax.ShapeDtypeStruct((NC, NP), jnp.float32),
    mesh=plsc.VectorSubcoreMesh(core_axis_name="c", subcore_axis_name="s"),
    scratch_types=[
        pltpu.VMEM_SHARED((NP,), jnp.float32),
        pltpu.VMEM_SHARED((NP,), jnp.float32),
        pltpu.VMEM((BLK,), jnp.int32),
        pltpu.VMEM((BLK,), jnp.int32),
        pltpu.VMEM((BLK,), jnp.int32),
        pltpu.VMEM((BLK,), jnp.int32),
        pltpu.VMEM((BLK,), jnp.float32),
        pltpu.VMEM((BLK,), jnp.float32),
        pltpu.VMEM((BLK,), jnp.float32),
        pltpu.VMEM((BLK,), jnp.float32),
        pltpu.VMEM((BLK,), jnp.float32),
        pltpu.VMEM((SUB,), jnp.float32),
        pltpu.SemaphoreType.DMA,
        pltpu.SemaphoreType.DMA,
        pltpu.SemaphoreType.DMA,
    ],
)


def _act_body(yp_ref, m_ref, d_ref, o_ref):
    y = yp_ref[0] + yp_ref[1]
    y = jnp.where(y >= THRESHOLD, y, 0.0)
    a = jnp.tanh(jnp.maximum(SLOPE * y, 0.0))
    o_ref[...] = jnp.where(m_ref[...] > 0.0, d_ref[...], a)


_act = pl.pallas_call(
    _act_body,
    out_shape=jax.ShapeDtypeStruct((NP // 128, 128), jnp.float32),
)


def kernel(inputs, values, rows, cols, sensory_idx):
    zeros = jnp.zeros((NP,), jnp.float32)
    mask = zeros.at[sensory_idx].set(1.0)
    drives = [zeros.at[sensory_idx].set(inputs[:, t]) for t in range(L)]

    pad_e = NNZ_P - NNZ
    cols_p = jnp.pad(cols, (0, pad_e))
    rows_p = jnp.pad(rows, (0, pad_e))
    vals_p = jnp.pad(values, (0, pad_e))
    m2 = mask.reshape(NP // 128, 128)

    x = drives[0]
    acts = [x]
    for t in range(1, L):
        yp = _spmv(x, cols_p, rows_p, vals_p)
        xn = _act(yp.reshape(NC, NP // 128, 128), m2,
                  drives[t].reshape(NP // 128, 128))
        x = xn.reshape(NP)
        acts.append(x)
    return jnp.stack([a[:N] for a in acts], axis=1)


# speedup vs baseline: 2.0736x; 1.0733x over previous
import jax
import jax.numpy as jnp
from jax import lax
from jax.experimental import pallas as pl
from jax.experimental.pallas import tpu as pltpu
from jax.experimental.pallas import tpu_sc as plsc

N = 100000
NNZ = 3200000
L = 5
THRESHOLD = 0.01
SLOPE = 5.0

NC, NS = 2, 16
NW = NC * NS
LANES = 16

NP = 100352
NBLK = 50
BLK = 2048
EPW = NBLK * BLK
NNZ_P = NW * EPW
SUB = NP // NS


def _spmv_body(x_hbm, cols_hbm, rows_hbm, vals_hbm, yp_hbm,
               x_sh, y_sh,
               cols_v0, cols_v1, rows_v0, rows_v1, vals_v0, vals_v1,
               xg_v0, xg_v1, w_v, zb_v, sem_in0, sem_in1, sem_g):
    cid = lax.axis_index("c")
    sid = lax.axis_index("s")
    wid = cid * NS + sid

    cols_v = (cols_v0, cols_v1)
    rows_v = (rows_v0, rows_v1)
    vals_v = (vals_v0, vals_v1)
    sem_in = (sem_in0, sem_in1)

    pltpu.sync_copy(x_hbm.at[pl.ds(sid * SUB, SUB)],
                    x_sh.at[pl.ds(sid * SUB, SUB)])

    @pl.loop(0, SUB, step=LANES)
    def _(i):
        zb_v[pl.ds(i, LANES)] = jnp.zeros((LANES,), jnp.float32)

    pltpu.sync_copy(zb_v, y_sh.at[pl.ds(sid * SUB, SUB)])
    plsc.subcore_barrier()

    e_base = wid * EPW

    def stage(slot, bb):
        e0 = e_base + bb * BLK
        pltpu.async_copy(cols_hbm.at[pl.ds(e0, BLK)], cols_v[slot], sem_in[slot])
        pltpu.async_copy(rows_hbm.at[pl.ds(e0, BLK)], rows_v[slot], sem_in[slot])
        pltpu.async_copy(vals_hbm.at[pl.ds(e0, BLK)], vals_v[slot], sem_in[slot])

    def drain_stage(slot, bb):
        e0 = e_base + bb * BLK
        pltpu.make_async_copy(cols_hbm.at[pl.ds(e0, BLK)], cols_v[slot], sem_in[slot]).wait()
        pltpu.make_async_copy(rows_hbm.at[pl.ds(e0, BLK)], rows_v[slot], sem_in[slot]).wait()
        pltpu.make_async_copy(vals_hbm.at[pl.ds(e0, BLK)], vals_v[slot], sem_in[slot]).wait()

    xg_v = (xg_v0, xg_v1)

    def wait_gather(slot):
        pltpu.make_async_copy(x_sh.at[cols_v[slot]], xg_v[slot], sem_g).wait()

    def fire_gather(slot):
        pltpu.make_async_copy(x_sh.at[cols_v[slot]], xg_v[slot], sem_g).start()

    def phase(cur, bb, do_next, do_stage2):
        nxt = 1 - cur
        wait_gather(cur)
        if do_next:
            drain_stage(nxt, bb + 1)
            fire_gather(nxt)

        @pl.loop(0, BLK, step=4 * LANES)
        def _(i):
            for u in range(4):
                o = u * LANES
                w_v[pl.ds(i + o, LANES)] = (
                    vals_v[cur][pl.ds(i + o, LANES)] * xg_v[cur][pl.ds(i + o, LANES)])

        pltpu.sync_copy(w_v, y_sh.at[rows_v[cur]], add=True)
        if do_stage2:
            stage(cur, bb + 2)

    stage(0, 0)
    stage(1, 1)
    drain_stage(0, 0)
    fire_gather(0)
    phase(0, 0, do_next=True, do_stage2=True)

    @pl.loop(1, NBLK - 3, step=2)
    def _(bb):
        phase(1, bb, do_next=True, do_stage2=True)
        phase(0, bb + 1, do_next=True, do_stage2=True)

    phase(1, NBLK - 3, do_next=True, do_stage2=True)
    phase(0, NBLK - 2, do_next=True, do_stage2=False)
    phase(1, NBLK - 1, do_next=False, do_stage2=False)

    plsc.subcore_barrier()
    pltpu.sync_copy(y_sh.at[pl.ds(sid * SUB, SUB)],
                    yp_hbm.at[cid, pl.ds(sid * SUB, SUB)])


_spmv = pl.kernel(
    _spmv_body,
    out_type=jax.ShapeDtypeStruct((NC, NP), jnp.float32),
    mesh=plsc.VectorSubcoreMesh(core_axis_name="c", subcore_axis_name="s"),
    scratch_types=[
        pltpu.VMEM_SHARED((NP,), jnp.float32),
        pltpu.VMEM_SHARED((NP,), jnp.float32),
        pltpu.VMEM((BLK,), jnp.int32),
        pltpu.VMEM((BLK,), jnp.int32),
        pltpu.VMEM((BLK,), jnp.int32),
        pltpu.VMEM((BLK,), jnp.int32),
        pltpu.VMEM((BLK,), jnp.float32),
        pltpu.VMEM((BLK,), jnp.float32),
        pltpu.VMEM((BLK,), jnp.float32),
        pltpu.VMEM((BLK,), jnp.float32),
        pltpu.VMEM((BLK,), jnp.float32),
        pltpu.VMEM((SUB,), jnp.float32),
        pltpu.SemaphoreType.DMA,
        pltpu.SemaphoreType.DMA,
        pltpu.SemaphoreType.DMA,
    ],
)


def _act_body(yp_ref, m_ref, d_ref, o_ref):
    y = yp_ref[0] + yp_ref[1]
    y = jnp.where(y >= THRESHOLD, y, 0.0)
    a = jnp.tanh(jnp.maximum(SLOPE * y, 0.0))
    o_ref[...] = jnp.where(m_ref[...] > 0.0, d_ref[...], a)


_act = pl.pallas_call(
    _act_body,
    out_shape=jax.ShapeDtypeStruct((NP // 128, 128), jnp.float32),
)


def kernel(inputs, values, rows, cols, sensory_idx):
    zeros = jnp.zeros((NP,), jnp.float32)
    mask = zeros.at[sensory_idx].set(1.0)
    drives = [zeros.at[sensory_idx].set(inputs[:, t]) for t in range(L)]

    pad_e = NNZ_P - NNZ
    cols_p = jnp.pad(cols, (0, pad_e))
    rows_p = jnp.pad(rows, (0, pad_e))
    vals_p = jnp.pad(values, (0, pad_e))
    m2 = mask.reshape(NP // 128, 128)

    x = drives[0]
    acts = [x]
    for t in range(1, L):
        yp = _spmv(x, cols_p, rows_p, vals_p)
        xn = _act(yp.reshape(NC, NP // 128, 128), m2,
                  drives[t].reshape(NP // 128, 128))
        x = xn.reshape(NP)
        acts.append(x)
    return jnp.stack([a[:N] for a in acts], axis=1)


# speedup vs baseline: 2.0755x; 1.0009x over previous
import jax
import jax.numpy as jnp
from jax import lax
from jax.experimental import pallas as pl
from jax.experimental.pallas import tpu as pltpu
from jax.experimental.pallas import tpu_sc as plsc

N = 100000
NNZ = 3200000
L = 5
THRESHOLD = 0.01
SLOPE = 5.0

NC, NS = 2, 16
NW = NC * NS
LANES = 16

NP = 100352
NBLK = 50
BLK = 2048
EPW = NBLK * BLK
NNZ_P = NW * EPW
SUB = NP // NS


def _spmv_body(x_hbm, cols_hbm, rows_hbm, vals_hbm, yp_hbm,
               x_sh, y_sh,
               cols_v0, cols_v1, rows_v0, rows_v1, vals_v0, vals_v1,
               xg_v0, xg_v1, w_v, zb_v, sem_in0, sem_in1, sem_g):
    cid = lax.axis_index("c")
    sid = lax.axis_index("s")
    wid = cid * NS + sid

    cols_v = (cols_v0, cols_v1)
    rows_v = (rows_v0, rows_v1)
    vals_v = (vals_v0, vals_v1)
    sem_in = (sem_in0, sem_in1)

    pltpu.sync_copy(x_hbm.at[pl.ds(sid * SUB, SUB)],
                    x_sh.at[pl.ds(sid * SUB, SUB)])

    @pl.loop(0, SUB, step=LANES)
    def _(i):
        zb_v[pl.ds(i, LANES)] = jnp.zeros((LANES,), jnp.float32)

    pltpu.sync_copy(zb_v, y_sh.at[pl.ds(sid * SUB, SUB)])
    plsc.subcore_barrier()

    e_base = wid * EPW

    def stage(slot, bb):
        e0 = e_base + bb * BLK
        pltpu.async_copy(cols_hbm.at[pl.ds(e0, BLK)], cols_v[slot], sem_in[slot])
        pltpu.async_copy(rows_hbm.at[pl.ds(e0, BLK)], rows_v[slot], sem_in[slot])
        pltpu.async_copy(vals_hbm.at[pl.ds(e0, BLK)], vals_v[slot], sem_in[slot])

    def drain_stage(slot, bb):
        e0 = e_base + bb * BLK
        pltpu.make_async_copy(cols_hbm.at[pl.ds(e0, BLK)], cols_v[slot], sem_in[slot]).wait()
        pltpu.make_async_copy(rows_hbm.at[pl.ds(e0, BLK)], rows_v[slot], sem_in[slot]).wait()
        pltpu.make_async_copy(vals_hbm.at[pl.ds(e0, BLK)], vals_v[slot], sem_in[slot]).wait()

    xg_v = (xg_v0, xg_v1)

    def wait_gather(slot):
        pltpu.make_async_copy(x_sh.at[cols_v[slot]], xg_v[slot], sem_g).wait()

    def fire_gather(slot):
        pltpu.make_async_copy(x_sh.at[cols_v[slot]], xg_v[slot], sem_g).start()

    def phase(cur, bb, do_next, do_stage2):
        nxt = 1 - cur
        wait_gather(cur)
        if do_next:
            drain_stage(nxt, bb + 1)
            fire_gather(nxt)

        @pl.loop(0, BLK, step=8 * LANES)
        def _(i):
            for u in range(8):
                o = u * LANES
                w_v[pl.ds(i + o, LANES)] = (
                    vals_v[cur][pl.ds(i + o, LANES)] * xg_v[cur][pl.ds(i + o, LANES)])

        pltpu.sync_copy(w_v, y_sh.at[rows_v[cur]], add=True)
        if do_stage2:
            stage(cur, bb + 2)

    stage(0, 0)
    stage(1, 1)
    drain_stage(0, 0)
    fire_gather(0)
    phase(0, 0, do_next=True, do_stage2=True)

    @pl.loop(1, NBLK - 3, step=2)
    def _(bb):
        phase(1, bb, do_next=True, do_stage2=True)
        phase(0, bb + 1, do_next=True, do_stage2=True)

    phase(1, NBLK - 3, do_next=True, do_stage2=True)
    phase(0, NBLK - 2, do_next=True, do_stage2=False)
    phase(1, NBLK - 1, do_next=False, do_stage2=False)

    plsc.subcore_barrier()
    pltpu.sync_copy(y_sh.at[pl.ds(sid * SUB, SUB)],
                    yp_hbm.at[cid, pl.ds(sid * SUB, SUB)])


_spmv = pl.kernel(
    _spmv_body,
    out_type=jax.ShapeDtypeStruct((NC, NP), jnp.float32),
    mesh=plsc.VectorSubcoreMesh(core_axis_name="c", subcore_axis_name="s"),
    scratch_types=[
        pltpu.VMEM_SHARED((NP,), jnp.float32),
        pltpu.VMEM_SHARED((NP,), jnp.float32),
        pltpu.VMEM((BLK,), jnp.int32),
        pltpu.VMEM((BLK,), jnp.int32),
        pltpu.VMEM((BLK,), jnp.int32),
        pltpu.VMEM((BLK,), jnp.int32),
        pltpu.VMEM((BLK,), jnp.float32),
        pltpu.VMEM((BLK,), jnp.float32),
        pltpu.VMEM((BLK,), jnp.float32),
        pltpu.VMEM((BLK,), jnp.float32),
        pltpu.VMEM((BLK,), jnp.float32),
        pltpu.VMEM((SUB,), jnp.float32),
        pltpu.SemaphoreType.DMA,
        pltpu.SemaphoreType.DMA,
        pltpu.SemaphoreType.DMA,
    ],
)


def _act_body(yp_ref, m_ref, d_ref, o_ref):
    y = yp_ref[0] + yp_ref[1]
    y = jnp.where(y >= THRESHOLD, y, 0.0)
    a = jnp.tanh(jnp.maximum(SLOPE * y, 0.0))
    o_ref[...] = jnp.where(m_ref[...] > 0.0, d_ref[...], a)


_act = pl.pallas_call(
    _act_body,
    out_shape=jax.ShapeDtypeStruct((NP // 128, 128), jnp.float32),
)


def kernel(inputs, values, rows, cols, sensory_idx):
    zeros = jnp.zeros((NP,), jnp.float32)
    mask = zeros.at[sensory_idx].set(1.0)
    drives = [zeros.at[sensory_idx].set(inputs[:, t]) for t in range(L)]

    pad_e = NNZ_P - NNZ
    cols_p = jnp.pad(cols, (0, pad_e))
    rows_p = jnp.pad(rows, (0, pad_e))
    vals_p = jnp.pad(values, (0, pad_e))
    m2 = mask.reshape(NP // 128, 128)

    x = drives[0]
    acts = [x]
    for t in range(1, L):
        yp = _spmv(x, cols_p, rows_p, vals_p)
        xn = _act(yp.reshape(NC, NP // 128, 128), m2,
                  drives[t].reshape(NP // 128, 128))
        x = xn.reshape(NP)
        acts.append(x)
    return jnp.stack([a[:N] for a in acts], axis=1)


# speedup vs baseline: 2.0764x; 1.0004x over previous
import jax
import jax.numpy as jnp
from jax import lax
from jax.experimental import pallas as pl
from jax.experimental.pallas import tpu as pltpu
from jax.experimental.pallas import tpu_sc as plsc

N = 100000
NNZ = 3200000
L = 5
THRESHOLD = 0.01
SLOPE = 5.0

NC, NS = 2, 16
NW = NC * NS
LANES = 16

NP = 100352
NBLK = 50
BLK = 2048
EPW = NBLK * BLK
NNZ_P = NW * EPW
SUB = NP // NS


def _spmv_body(x_hbm, cols_hbm, rows_hbm, vals_hbm, yp_hbm,
               x_sh, y_sh,
               cols_v0, cols_v1, rows_v0, rows_v1, vals_v0, vals_v1,
               xg_v0, xg_v1, w_v, zb_v, sem_in0, sem_in1, sem_g):
    cid = lax.axis_index("c")
    sid = lax.axis_index("s")
    wid = cid * NS + sid

    cols_v = (cols_v0, cols_v1)
    rows_v = (rows_v0, rows_v1)
    vals_v = (vals_v0, vals_v1)
    sem_in = (sem_in0, sem_in1)

    pltpu.sync_copy(x_hbm.at[pl.ds(sid * SUB, SUB)],
                    x_sh.at[pl.ds(sid * SUB, SUB)])

    @pl.loop(0, SUB, step=LANES)
    def _(i):
        zb_v[pl.ds(i, LANES)] = jnp.zeros((LANES,), jnp.float32)

    pltpu.sync_copy(zb_v, y_sh.at[pl.ds(sid * SUB, SUB)])
    plsc.subcore_barrier()

    e_base = wid * EPW

    def stage(slot, bb):
        e0 = e_base + bb * BLK
        pltpu.async_copy(cols_hbm.at[pl.ds(e0, BLK)], cols_v[slot], sem_in[slot])
        pltpu.async_copy(rows_hbm.at[pl.ds(e0, BLK)], rows_v[slot], sem_in[slot])
        pltpu.async_copy(vals_hbm.at[pl.ds(e0, BLK)], vals_v[slot], sem_in[slot])

    def drain_stage(slot, bb):
        e0 = e_base + bb * BLK
        pltpu.make_async_copy(cols_hbm.at[pl.ds(e0, BLK)], cols_v[slot], sem_in[slot]).wait()
        pltpu.make_async_copy(rows_hbm.at[pl.ds(e0, BLK)], rows_v[slot], sem_in[slot]).wait()
        pltpu.make_async_copy(vals_hbm.at[pl.ds(e0, BLK)], vals_v[slot], sem_in[slot]).wait()

    xg_v = (xg_v0, xg_v1)

    def wait_gather(slot):
        pltpu.make_async_copy(x_sh.at[cols_v[slot]], xg_v[slot], sem_g).wait()

    def fire_gather(slot):
        pltpu.make_async_copy(x_sh.at[cols_v[slot]], xg_v[slot], sem_g).start()

    def phase(cur, bb, do_next, do_stage2):
        nxt = 1 - cur
        wait_gather(cur)
        if do_next:
            drain_stage(nxt, bb + 1)
            fire_gather(nxt)

        @pl.loop(0, BLK, step=4 * LANES)
        def _(i):
            for u in range(4):
                o = u * LANES
                w_v[pl.ds(i + o, LANES)] = (
                    vals_v[cur][pl.ds(i + o, LANES)] * xg_v[cur][pl.ds(i + o, LANES)])

        pltpu.sync_copy(w_v, y_sh.at[rows_v[cur]], add=True)
        if do_stage2:
            stage(cur, bb + 2)

    stage(0, 0)
    stage(1, 1)
    drain_stage(0, 0)
    fire_gather(0)
    phase(0, 0, do_next=True, do_stage2=True)

    @pl.loop(1, NBLK - 3, step=2)
    def _(bb):
        phase(1, bb, do_next=True, do_stage2=True)
        phase(0, bb + 1, do_next=True, do_stage2=True)

    phase(1, NBLK - 3, do_next=True, do_stage2=True)
    phase(0, NBLK - 2, do_next=True, do_stage2=False)
    phase(1, NBLK - 1, do_next=False, do_stage2=False)

    plsc.subcore_barrier()
    pltpu.sync_copy(y_sh.at[pl.ds(sid * SUB, SUB)],
                    yp_hbm.at[cid, pl.ds(sid * SUB, SUB)])


_spmv = pl.kernel(
    _spmv_body,
    out_type=jax.ShapeDtypeStruct((NC, NP), jnp.float32),
    mesh=plsc.VectorSubcoreMesh(core_axis_name="c", subcore_axis_name="s"),
    scratch_types=[
        pltpu.VMEM_SHARED((NP,), jnp.float32),
        pltpu.VMEM_SHARED((NP,), jnp.float32),
        pltpu.VMEM((BLK,), jnp.int32),
        pltpu.VMEM((BLK,), jnp.int32),
        pltpu.VMEM((BLK,), jnp.int32),
        pltpu.VMEM((BLK,), jnp.int32),
        pltpu.VMEM((BLK,), jnp.float32),
        pltpu.VMEM((BLK,), jnp.float32),
        pltpu.VMEM((BLK,), jnp.float32),
        pltpu.VMEM((BLK,), jnp.float32),
        pltpu.VMEM((BLK,), jnp.float32),
        pltpu.VMEM((SUB,), jnp.float32),
        pltpu.SemaphoreType.DMA,
        pltpu.SemaphoreType.DMA,
        pltpu.SemaphoreType.DMA,
    ],
)


def _act_body(yp_ref, m_ref, d_ref, o_ref):
    y = yp_ref[0] + yp_ref[1]
    y = jnp.where(y >= THRESHOLD, y, 0.0)
    a = jnp.tanh(jnp.maximum(SLOPE * y, 0.0))
    o_ref[...] = jnp.where(m_ref[...] > 0.0, d_ref[...], a)


_act = pl.pallas_call(
    _act_body,
    out_shape=jax.ShapeDtypeStruct((NP // 128, 128), jnp.float32),
)


def kernel(inputs, values, rows, cols, sensory_idx):
    zeros = jnp.zeros((NP,), jnp.float32)
    mask = zeros.at[sensory_idx].set(1.0)
    drives = [zeros.at[sensory_idx].set(inputs[:, t]) for t in range(L)]

    pad_e = NNZ_P - NNZ
    cols_p = jnp.pad(cols, (0, pad_e))
    rows_p = jnp.pad(rows, (0, pad_e))
    vals_p = jnp.pad(values, (0, pad_e))
    m2 = mask.reshape(NP // 128, 128)

    x = drives[0]
    acts = [x]
    for t in range(1, L):
        yp = _spmv(x, cols_p, rows_p, vals_p)
        xn = _act(yp.reshape(NC, NP // 128, 128), m2,
                  drives[t].reshape(NP // 128, 128))
        x = xn.reshape(NP)
        acts.append(x)
    return jnp.stack([a[:N] for a in acts], axis=1)
